# baseline (speedup 1.0000x reference)
import functools

import jax
import jax.numpy as jnp
from jax import lax
from jax.experimental import pallas as pl
from jax.experimental.pallas import tpu as pltpu
from jax.experimental.pallas import tpu_sc as plsc

_N = 50000
_B = 16
_NPAD = 51200
_RPT = _NPAD // 16
_CB = 320
_NCB = _RPT // _CB
_TB = 320
_NTB = _RPT // _TB
_E6 = 6 * 800000
_SUP = 512
_W = 128
_K = _SUP // _W
_NT = 32
_EPAD = ((_E6 + _NT * _SUP - 1) // (_NT * _SUP)) * (_NT * _SUP)
_EPT = _EPAD // _NT
_NSUP = _EPT // _SUP
_STEPS = 6
_MB = _NPAD // 8


def _prop_body(u_hbm, avec_hbm, rows_hbm, cols_hbm, vals_hbm, eflag_hbm,
               bflag_hbm, pout_hbm, pbn_hbm, h_s, acc_s, gath0, gath1,
               rows_a, rows_b, cols_a, cols_b, vals_a, vals_b, tbuf, avv,
               mark_v, probe_v, done_v, sem_l, sem_g, sem_s):
    gath = [gath0, gath1]
    rows = [rows_a, rows_b]
    cols = [cols_a, cols_b]
    vals = [vals_a, vals_b]
    c = lax.axis_index("c")
    s = lax.axis_index("s")
    oc = 1 - c
    t = c * 16 + s
    row0 = s * _RPT
    wbase0 = t * (_EPT // _W)
    ebase0 = t * _EPT
    iota16 = lax.iota(jnp.int32, 16)

    b_p0 = gath0.at[pl.ds(0, _CB)]
    b_p1 = gath1.at[pl.ds(0, _CB)]
    b_h = b_p0

    mark_v[...] = jnp.ones((16,), jnp.float32)
    pltpu.sync_copy(avec_hbm, avv)

    def poll(flag_slice):
        done_v[0] = 0

        def outer(_, __):
            @pl.when(done_v[0] == 0)
            def _():
                def inner(_, __):
                    @pl.when(done_v[0] == 0)
                    def _():
                        pltpu.sync_copy(flag_slice, probe_v)
                        done_v[0] = (jnp.sum(probe_v[...])
                                     != 0.0).astype(jnp.int32)
                    return 0

                lax.fori_loop(0, 32, inner, 0)
            return 0

        lax.fori_loop(0, 256, outer, 0)

    def node_vec(buf, n):
        return plsc.load_gather(buf, [iota16, jnp.full((16,), n, jnp.int32)])

    def fire_lin(i, b):
        pltpu.async_copy(rows_hbm.at[pl.ds(wbase0 + i * _K, _K)],
                         rows[b], sem_l)
        pltpu.async_copy(cols_hbm.at[pl.ds(wbase0 + i * _K, _K)],
                         cols[b], sem_l)
        pltpu.async_copy(vals_hbm.at[pl.ds(ebase0 + i * _SUP, _SUP)],
                         vals[b], sem_l)

    def wait_lin(i, b):
        pltpu.make_async_copy(rows_hbm.at[pl.ds(wbase0 + i * _K, _K)],
                              rows[b], sem_l).wait()
        pltpu.make_async_copy(cols_hbm.at[pl.ds(wbase0 + i * _K, _K)],
                              cols[b], sem_l).wait()
        pltpu.make_async_copy(vals_hbm.at[pl.ds(ebase0 + i * _SUP, _SUP)],
                              vals[b], sem_l).wait()

    def fire_gath(b):
        for j in range(_K):
            pltpu.async_copy(h_s.at[rows[b].at[j]],
                             gath[b].at[pl.ds(j * _W, _W)], sem_g)

    def fire_scat(b):
        for j in range(_K):
            pltpu.async_copy(gath[b].at[pl.ds(j * _W, _W)],
                             acc_s.at[cols[b].at[j]], sem_s, add=True)

    def drain(b, sem):
        for j in range(_K):
            pltpu.make_async_copy(gath[b].at[pl.ds(j * _W, _W)],
                                  acc_s.at[cols[b].at[j]], sem).wait()

    def scale_chunk(b):
        g = gath[b]
        v = vals[b]

        def scale(gi, _):
            base = gi * 16
            v16 = v[pl.ds(base, 16)]
            for l in range(16):
                g[base + l, :] = g[base + l, :] * v16[l]
            return 0

        lax.fori_loop(0, _SUP // 16, scale, 0)

    def process(b):
        drain(b, sem_g)
        scale_chunk(b)
        fire_scat(b)

    def step(k, _):
        @pl.when(k == 0)
        def _():
            for cb in range(_NCB):
                base = row0 + cb * _CB
                pltpu.sync_copy(u_hbm.at[:, pl.ds(base, _CB)], tbuf)

                def stage(n, _):
                    b_h[n, :] = node_vec(tbuf, n)
                    return 0

                lax.fori_loop(0, _CB, stage, 0, unroll=8)
                pltpu.sync_copy(b_h, h_s.at[pl.ds(base, _CB)])

        @pl.when(k > 0)
        def _():
            poll(eflag_hbm.at[oc, k - 1])
            av = avv[2 * k, :]
            amv = avv[2 * k + 1, :]
            for cb in range(_NCB):
                base = row0 + cb * _CB
                pltpu.sync_copy(u_hbm.at[:, pl.ds(base, _CB)], tbuf)
                pltpu.sync_copy(pout_hbm.at[0, pl.ds(base, _CB)], b_p0)
                pltpu.sync_copy(pout_hbm.at[1, pl.ds(base, _CB)], b_p1)

                def blend(n, _):
                    b_h[n, :] = (node_vec(tbuf, n) * av
                                 + (b_p0[n, :] + b_p1[n, :]) * amv)
                    return 0

                lax.fori_loop(0, _CB, blend, 0, unroll=8)
                pltpu.sync_copy(b_h, h_s.at[pl.ds(base, _CB)])
            plsc.subcore_barrier()

            @pl.when(s == 0)
            def _():
                pltpu.sync_copy(mark_v, bflag_hbm.at[c, k])

        def zero(i, _):
            b_p1[i, :] = jnp.zeros((16,), jnp.float32)
            return 0

        lax.fori_loop(0, _CB, zero, 0, unroll=8)
        for cb in range(_NCB):
            pltpu.sync_copy(b_p1, acc_s.at[pl.ds(row0 + cb * _CB, _CB)])
        plsc.subcore_barrier()

        fire_lin(0, 0)
        wait_lin(0, 0)
        fire_gath(0)
        fire_lin(1, 1)
        process(0)
        wait_lin(1, 1)
        fire_gath(1)

        def pair(pi, _):
            i = 1 + 2 * pi
            for b in (1, 0):
                drain(1 - b, sem_s)
                fire_lin(i + 1, 1 - b)
                process(b)
                wait_lin(i + 1, 1 - b)
                fire_gath(1 - b)
                i = i + 1
            return 0

        lax.fori_loop(0, (_NSUP - 2) // 2, pair, 0)

        drain(0, sem_s)
        process(1)
        drain(1, sem_s)
        plsc.subcore_barrier()

        @pl.when(k < _STEPS - 1)
        def _():
            @pl.when(k > 0)
            def _():
                poll(bflag_hbm.at[oc, k])

            pltpu.sync_copy(acc_s.at[pl.ds(row0, _RPT)],
                            pout_hbm.at[c, pl.ds(row0, _RPT)])
            plsc.subcore_barrier()

            @pl.when(s == 0)
            def _():
                pltpu.sync_copy(mark_v, eflag_hbm.at[c, k])

        @pl.when(k == _STEPS - 1)
        def _():
            for cb in range(_NTB):
                base = row0 + cb * _TB
                pltpu.sync_copy(acc_s.at[pl.ds(base, _TB)],
                                gath0.at[pl.ds(0, _TB)])

                def xpose(nb, _):
                    n0 = nb * 16
                    for f in range(16):
                        fv = plsc.load_gather(
                            gath0, [iota16 + n0,
                                    jnp.full((16,), f, jnp.int32)])
                        tbuf[f, pl.ds(n0, 16)] = fv
                    return 0

                lax.fori_loop(0, _TB // 16, xpose, 0)
                pltpu.sync_copy(tbuf, pbn_hbm.at[c, :, pl.ds(base, _TB)])

        return 0

    lax.fori_loop(0, _STEPS, step, 0)


_prop_call = pl.kernel(
    _prop_body,
    out_type=[jax.ShapeDtypeStruct((2, _NPAD, _B), jnp.float32),
              jax.ShapeDtypeStruct((2, _B, _NPAD), jnp.float32)],
    mesh=plsc.VectorSubcoreMesh(core_axis_name="c", subcore_axis_name="s"),
    compiler_params=pltpu.CompilerParams(use_tc_tiling_on_sc=False,
                                         needs_layout_passes=False),
    scratch_types=[
        pltpu.VMEM_SHARED((_NPAD, _B), jnp.float32),
        pltpu.VMEM_SHARED((_NPAD, _B), jnp.float32),
        pltpu.VMEM((_SUP, _B), jnp.float32),
        pltpu.VMEM((_SUP, _B), jnp.float32),
        pltpu.VMEM((_K, _W), jnp.int32),
        pltpu.VMEM((_K, _W), jnp.int32),
        pltpu.VMEM((_K, _W), jnp.int32),
        pltpu.VMEM((_K, _W), jnp.int32),
        pltpu.VMEM((_SUP,), jnp.float32),
        pltpu.VMEM((_SUP,), jnp.float32),
        pltpu.VMEM((16, _TB), jnp.float32),
        pltpu.VMEM((2 * _STEPS, 16), jnp.float32),
        pltpu.VMEM((16,), jnp.float32),
        pltpu.VMEM((16,), jnp.float32),
        pltpu.SMEM((1,), jnp.int32),
        pltpu.SemaphoreType.DMA,
        pltpu.SemaphoreType.DMA,
        pltpu.SemaphoreType.DMA,
    ],
    name="prop_sc",
)


def _zero_body(e_ref, b_ref):
    e_ref[...] = jnp.zeros_like(e_ref)
    b_ref[...] = jnp.zeros_like(b_ref)


_zero_call = pl.pallas_call(
    _zero_body,
    out_shape=[jax.ShapeDtypeStruct((2, _STEPS, 16), jnp.float32),
               jax.ShapeDtypeStruct((2, _STEPS, 16), jnp.float32)],
    name="zero_flags_tc",
)


_NWG = 800000 // _W
_NWPAD = (_EPAD - _E6) // _W
_PAD_ROW = _N + 1
_PAD_COL = _N + 5


def _cat_body(r0, r1, r2, r3, r4, r5, c0, c1, c2, c3, c4, c5,
              v0, v1, v2, v3, v4, v5, rows_out, cols_out, vals_out,
              cbuf, sem):
    descs = []
    for g, (r, c, v) in enumerate([(r0, c0, v0), (r1, c1, v1),
                                   (r2, c2, v2), (r3, c3, v3),
                                   (r4, c4, v4), (r5, c5, v5)]):
        descs.append(pltpu.make_async_copy(
            r, rows_out.at[pl.ds(g * _NWG, _NWG)], sem))
        descs.append(pltpu.make_async_copy(
            c, cols_out.at[pl.ds(g * _NWG, _NWG)], sem))
        descs.append(pltpu.make_async_copy(
            v, vals_out.at[pl.ds(g * 800000, 800000)], sem))
    for d in descs:
        d.start()
    cbuf[...] = jnp.full((_NWPAD, _W), _PAD_ROW, jnp.int32)
    pltpu.sync_copy(cbuf, rows_out.at[pl.ds(6 * _NWG, _NWPAD)])
    cbuf[...] = jnp.full((_NWPAD, _W), _PAD_COL, jnp.int32)
    pltpu.sync_copy(cbuf, cols_out.at[pl.ds(6 * _NWG, _NWPAD)])
    for d in descs:
        d.wait()


_cat_call = pl.pallas_call(
    _cat_body,
    out_shape=[jax.ShapeDtypeStruct((_EPAD // _W, _W), jnp.int32),
               jax.ShapeDtypeStruct((_EPAD // _W, _W), jnp.int32),
               jax.ShapeDtypeStruct((_EPAD,), jnp.float32)],
    in_specs=[pl.BlockSpec(memory_space=pl.ANY)] * 18,
    out_specs=[pl.BlockSpec(memory_space=pl.ANY)] * 3,
    scratch_shapes=[pltpu.VMEM((_NWPAD, _W), jnp.int32),
                    pltpu.SemaphoreType.DMA],
    name="concat_edges_tc",
)


def _mlp_body(ctl_ref, u_ref, p0_ref, p1_ref, win_ref, bin_ref, wout_ref,
              par_ref, bias_ref, out_ref):
    a5 = par_ref[0]
    am5 = par_ref[1]
    ctl = ctl_ref[...]
    u = u_ref[...]
    h = u * a5 + (p0_ref[...] + p1_ref[...]) * am5
    acc = jnp.zeros_like(ctl)
    for j in range(64):
        hh = jnp.maximum(
            ctl * win_ref[0, j] + u * win_ref[1, j] + h * win_ref[2, j]
            + bin_ref[j], 0.0)
        acc = acc + hh * wout_ref[j]
    out_ref[...] = acc + bias_ref[...]


_mlp_call = pl.pallas_call(
    _mlp_body,
    out_shape=jax.ShapeDtypeStruct((_B, _NPAD), jnp.float32),
    grid=(_NPAD // _MB,),
    in_specs=[
        pl.BlockSpec((_B, _MB), lambda i: (0, i)),
        pl.BlockSpec((_B, _MB), lambda i: (0, i)),
        pl.BlockSpec((_B, _MB), lambda i: (0, i)),
        pl.BlockSpec((_B, _MB), lambda i: (0, i)),
        pl.BlockSpec(memory_space=pltpu.SMEM),
        pl.BlockSpec(memory_space=pltpu.SMEM),
        pl.BlockSpec(memory_space=pltpu.SMEM),
        pl.BlockSpec(memory_space=pltpu.SMEM),
        pl.BlockSpec((_B, 1), lambda i: (0, 0)),
    ],
    out_specs=pl.BlockSpec((_B, _MB), lambda i: (0, i)),
    name="mlp_head_tc",
)


def kernel(ctl_base, u_raw, cell_idx, rows_tfp, cols_tfp, vals_tfp,
           rows_tfn, cols_tfn, vals_tfn, rows_ppp, cols_ppp, vals_ppp,
           rows_ppn, cols_ppn, vals_ppn, rows_und, cols_und, vals_und,
           rows_mir, cols_mir, vals_mir, g_tf_pos, g_tf_neg, g_ppi_pos,
           g_ppi_neg, g_undir, g_mirna_neg, alpha_logits, cell_emb,
           W_in, b_in, W_out, b_out):
    sp = jax.nn.softplus
    gains = [sp(g_tf_pos), -sp(g_tf_neg), sp(g_ppi_pos), -sp(g_ppi_neg),
             sp(g_undir), -sp(g_mirna_neg)]
    alphas = jax.nn.sigmoid(alpha_logits)

    rows_all, cols_all, vals_all = _cat_call(
        rows_tfp.reshape(_NWG, _W), rows_tfn.reshape(_NWG, _W),
        rows_ppp.reshape(_NWG, _W), rows_ppn.reshape(_NWG, _W),
        rows_und.reshape(_NWG, _W), rows_mir.reshape(_NWG, _W),
        cols_tfp.reshape(_NWG, _W), cols_tfn.reshape(_NWG, _W),
        cols_ppp.reshape(_NWG, _W), cols_ppn.reshape(_NWG, _W),
        cols_und.reshape(_NWG, _W), cols_mir.reshape(_NWG, _W),
        gains[0] * vals_tfp, gains[1] * vals_tfn, gains[2] * vals_ppp,
        gains[3] * vals_ppn, gains[4] * vals_und, gains[5] * vals_mir)

    u_pad = jnp.pad(u_raw, ((0, 0), (0, _NPAD - _N)))
    ctl_pad = jnp.pad(ctl_base, ((0, 0), (0, _NPAD - _N)))

    ab = jnp.concatenate([jnp.ones((1,), jnp.float32), alphas[:_STEPS - 1]])
    avec = jnp.stack([jnp.stack([jnp.full((16,), ab[k], jnp.float32),
                                 jnp.full((16,), 1.0 - ab[k], jnp.float32)])
                      for k in range(_STEPS)]).reshape(2 * _STEPS, 16)

    eflag, bflag = _zero_call()
    _, pbn = _prop_call(u_pad, avec, rows_all, cols_all, vals_all,
                        eflag, bflag)

    a5 = alphas[5]
    par = jnp.stack([a5, 1.0 - a5])
    bias_b = cell_emb[cell_idx] @ W_out[:, 0] + b_out[0]
    y = _mlp_call(ctl_pad, u_pad, pbn[0], pbn[1],
                  W_in, b_in, W_out[:, 0], par, bias_b[:, None])
    return y[:, :_N]


# speedup vs baseline: 1.5207x; 1.5207x over previous
import functools

import jax
import jax.numpy as jnp
from jax import lax
from jax.experimental import pallas as pl
from jax.experimental.pallas import tpu as pltpu
from jax.experimental.pallas import tpu_sc as plsc

_N = 50000
_B = 16
_NPAD = 51200
_RPT = _NPAD // 16
_CB = 320
_NCB = _RPT // _CB
_TB = 320
_NTB = _RPT // _TB
_E6 = 6 * 800000
_SUP = 512
_W = 128
_K = _SUP // _W
_NT = 32
_EPAD = ((_E6 + _NT * _SUP - 1) // (_NT * _SUP)) * (_NT * _SUP)
_EPT = _EPAD // _NT
_NSUP = _EPT // _SUP
_STEPS = 6
_MB = _NPAD // 8


def _prop_body(u_hbm, avec_hbm, rows_hbm, cols_hbm, vals_hbm, eflag_hbm,
               bflag_hbm, pout_hbm, pbn_hbm, h_s, acc_s, gath0, gath1,
               rows_a, rows_b, cols_a, cols_b, vals_a, vals_b, tbuf, avv,
               mark_v, probe_v, done_v, sem_l, sem_g, sem_s):
    gath = [gath0, gath1]
    rows = [rows_a, rows_b]
    cols = [cols_a, cols_b]
    vals = [vals_a, vals_b]
    c = lax.axis_index("c")
    s = lax.axis_index("s")
    oc = 1 - c
    t = c * 16 + s
    row0 = s * _RPT
    wbase0 = t * (_EPT // _W)
    ebase0 = t * _EPT
    iota16 = lax.iota(jnp.int32, 16)

    b_p0 = gath0.at[pl.ds(0, _CB)]
    b_p1 = gath1.at[pl.ds(0, _CB)]
    b_h = b_p0

    mark_v[...] = jnp.ones((16,), jnp.float32)
    pltpu.sync_copy(avec_hbm, avv)

    def poll(flag_slice):
        done_v[0] = 0

        def outer(_, __):
            @pl.when(done_v[0] == 0)
            def _():
                def inner(_, __):
                    @pl.when(done_v[0] == 0)
                    def _():
                        pltpu.sync_copy(flag_slice, probe_v)
                        done_v[0] = (jnp.sum(probe_v[...])
                                     != 0.0).astype(jnp.int32)
                    return 0

                lax.fori_loop(0, 32, inner, 0)
            return 0

        lax.fori_loop(0, 256, outer, 0)

    def node_vec(buf, n):
        return plsc.load_gather(buf, [iota16, jnp.full((16,), n, jnp.int32)])

    def fire_lin(i, b):
        pltpu.async_copy(rows_hbm.at[pl.ds(wbase0 + i * _K, _K)],
                         rows[b], sem_l)
        pltpu.async_copy(cols_hbm.at[pl.ds(wbase0 + i * _K, _K)],
                         cols[b], sem_l)
        pltpu.async_copy(vals_hbm.at[pl.ds(ebase0 + i * _SUP, _SUP)],
                         vals[b], sem_l)

    def wait_lin(i, b):
        pltpu.make_async_copy(rows_hbm.at[pl.ds(wbase0 + i * _K, _K)],
                              rows[b], sem_l).wait()
        pltpu.make_async_copy(cols_hbm.at[pl.ds(wbase0 + i * _K, _K)],
                              cols[b], sem_l).wait()
        pltpu.make_async_copy(vals_hbm.at[pl.ds(ebase0 + i * _SUP, _SUP)],
                              vals[b], sem_l).wait()

    def fire_gath(b):
        for j in range(_K):
            pltpu.async_copy(h_s.at[rows[b].at[j]],
                             gath[b].at[pl.ds(j * _W, _W)], sem_g)

    def fire_scat(b):
        for j in range(_K):
            pltpu.async_copy(gath[b].at[pl.ds(j * _W, _W)],
                             acc_s.at[cols[b].at[j]], sem_s, add=True)

    def drain(b, sem):
        for j in range(_K):
            pltpu.make_async_copy(gath[b].at[pl.ds(j * _W, _W)],
                                  acc_s.at[cols[b].at[j]], sem).wait()

    def scale_chunk(b):
        g = gath[b]
        v = vals[b]

        def scale(gi, _):
            base = gi * 16
            v16 = v[pl.ds(base, 16)]
            for l in range(16):
                g[base + l, :] = g[base + l, :] * v16[l]
            return 0

        lax.fori_loop(0, _SUP // 16, scale, 0)

    def process(b):
        drain(b, sem_g)
        scale_chunk(b)
        fire_scat(b)

    def step(k, _):
        @pl.when(k == 0)
        def _():
            for cb in range(_NCB):
                base = row0 + cb * _CB
                pltpu.sync_copy(u_hbm.at[:, pl.ds(base, _CB)], tbuf)

                def stage(n, _):
                    b_h[n, :] = node_vec(tbuf, n)
                    return 0

                lax.fori_loop(0, _CB, stage, 0, unroll=8)
                pltpu.sync_copy(b_h, h_s.at[pl.ds(base, _CB)])

        @pl.when(k > 0)
        def _():
            poll(eflag_hbm.at[oc, k - 1])
            av = avv[2 * k, :]
            amv = avv[2 * k + 1, :]
            for cb in range(_NCB):
                base = row0 + cb * _CB
                pltpu.sync_copy(u_hbm.at[:, pl.ds(base, _CB)], tbuf)
                pltpu.sync_copy(pout_hbm.at[0, pl.ds(base, _CB)], b_p0)
                pltpu.sync_copy(pout_hbm.at[1, pl.ds(base, _CB)], b_p1)

                def blend(n, _):
                    b_h[n, :] = (node_vec(tbuf, n) * av
                                 + (b_p0[n, :] + b_p1[n, :]) * amv)
                    return 0

                lax.fori_loop(0, _CB, blend, 0, unroll=8)
                pltpu.sync_copy(b_h, h_s.at[pl.ds(base, _CB)])
            plsc.subcore_barrier()

            @pl.when(s == 0)
            def _():
                pltpu.sync_copy(mark_v, bflag_hbm.at[c, k])

        def zero(i, _):
            b_p1[i, :] = jnp.zeros((16,), jnp.float32)
            return 0

        lax.fori_loop(0, _CB, zero, 0, unroll=8)
        for cb in range(_NCB):
            pltpu.sync_copy(b_p1, acc_s.at[pl.ds(row0 + cb * _CB, _CB)])
        plsc.subcore_barrier()

        fire_lin(0, 0)
        wait_lin(0, 0)
        fire_gath(0)
        fire_lin(1, 1)
        process(0)
        wait_lin(1, 1)
        fire_gath(1)

        def pair(pi, _):
            i = 1 + 2 * pi
            for b in (1, 0):
                drain(1 - b, sem_s)
                fire_lin(i + 1, 1 - b)
                process(b)
                wait_lin(i + 1, 1 - b)
                fire_gath(1 - b)
                i = i + 1
            return 0

        lax.fori_loop(0, (_NSUP - 2) // 2, pair, 0)

        drain(0, sem_s)
        process(1)
        drain(1, sem_s)
        plsc.subcore_barrier()

        @pl.when(k < _STEPS - 1)
        def _():
            @pl.when(k > 0)
            def _():
                poll(bflag_hbm.at[oc, k])

            pltpu.sync_copy(acc_s.at[pl.ds(row0, _RPT)],
                            pout_hbm.at[c, pl.ds(row0, _RPT)])
            plsc.subcore_barrier()

            @pl.when(s == 0)
            def _():
                pltpu.sync_copy(mark_v, eflag_hbm.at[c, k])

        @pl.when(k == _STEPS - 1)
        def _():
            for cb in range(_NTB):
                base = row0 + cb * _TB
                pltpu.sync_copy(acc_s.at[pl.ds(base, _TB)],
                                gath0.at[pl.ds(0, _TB)])

                def xpose(nb, _):
                    n0 = nb * 16
                    for f in range(16):
                        fv = plsc.load_gather(
                            gath0, [iota16 + n0,
                                    jnp.full((16,), f, jnp.int32)])
                        tbuf[f, pl.ds(n0, 16)] = fv
                    return 0

                lax.fori_loop(0, _TB // 16, xpose, 0)
                pltpu.sync_copy(tbuf, pbn_hbm.at[c, :, pl.ds(base, _TB)])

        return 0

    lax.fori_loop(0, _STEPS, step, 0)


_prop_call = pl.kernel(
    _prop_body,
    out_type=[jax.ShapeDtypeStruct((2, _NPAD, _B), jnp.float32),
              jax.ShapeDtypeStruct((2, _B, _NPAD), jnp.float32)],
    mesh=plsc.VectorSubcoreMesh(core_axis_name="c", subcore_axis_name="s"),
    compiler_params=pltpu.CompilerParams(use_tc_tiling_on_sc=False,
                                         needs_layout_passes=False),
    scratch_types=[
        pltpu.VMEM_SHARED((_NPAD, _B), jnp.float32),
        pltpu.VMEM_SHARED((_NPAD, _B), jnp.float32),
        pltpu.VMEM((_SUP, _B), jnp.float32),
        pltpu.VMEM((_SUP, _B), jnp.float32),
        pltpu.VMEM((_K, _W), jnp.int32),
        pltpu.VMEM((_K, _W), jnp.int32),
        pltpu.VMEM((_K, _W), jnp.int32),
        pltpu.VMEM((_K, _W), jnp.int32),
        pltpu.VMEM((_SUP,), jnp.float32),
        pltpu.VMEM((_SUP,), jnp.float32),
        pltpu.VMEM((16, _TB), jnp.float32),
        pltpu.VMEM((2 * _STEPS, 16), jnp.float32),
        pltpu.VMEM((16,), jnp.float32),
        pltpu.VMEM((16,), jnp.float32),
        pltpu.SMEM((1,), jnp.int32),
        pltpu.SemaphoreType.DMA,
        pltpu.SemaphoreType.DMA,
        pltpu.SemaphoreType.DMA,
    ],
    name="prop_sc",
)


def _zero_body(e_ref, b_ref):
    e_ref[...] = jnp.zeros_like(e_ref)
    b_ref[...] = jnp.zeros_like(b_ref)


_zero_call = pl.pallas_call(
    _zero_body,
    out_shape=[jax.ShapeDtypeStruct((2, _STEPS, 16), jnp.float32),
               jax.ShapeDtypeStruct((2, _STEPS, 16), jnp.float32)],
    name="zero_flags_tc",
)


_NWG = 800000 // _W
_NWPAD = (_EPAD - _E6) // _W
_PAD_ROW = _N + 1
_PAD_COL = _N + 5


def _mlp_body(ctl_ref, u_ref, p0_ref, p1_ref, win_ref, bin_ref, wout_ref,
              par_ref, bias_ref, out_ref):
    a5 = par_ref[0]
    am5 = par_ref[1]
    ctl = ctl_ref[...]
    u = u_ref[...]
    h = u * a5 + (p0_ref[...] + p1_ref[...]) * am5
    acc = jnp.zeros_like(ctl)
    for j in range(64):
        hh = jnp.maximum(
            ctl * win_ref[0, j] + u * win_ref[1, j] + h * win_ref[2, j]
            + bin_ref[j], 0.0)
        acc = acc + hh * wout_ref[j]
    out_ref[...] = acc + bias_ref[...]


_mlp_call = pl.pallas_call(
    _mlp_body,
    out_shape=jax.ShapeDtypeStruct((_B, _NPAD), jnp.float32),
    grid=(_NPAD // _MB,),
    in_specs=[
        pl.BlockSpec((_B, _MB), lambda i: (0, i)),
        pl.BlockSpec((_B, _MB), lambda i: (0, i)),
        pl.BlockSpec((_B, _MB), lambda i: (0, i)),
        pl.BlockSpec((_B, _MB), lambda i: (0, i)),
        pl.BlockSpec(memory_space=pltpu.SMEM),
        pl.BlockSpec(memory_space=pltpu.SMEM),
        pl.BlockSpec(memory_space=pltpu.SMEM),
        pl.BlockSpec(memory_space=pltpu.SMEM),
        pl.BlockSpec((_B, 1), lambda i: (0, 0)),
    ],
    out_specs=pl.BlockSpec((_B, _MB), lambda i: (0, i)),
    name="mlp_head_tc",
)


def kernel(ctl_base, u_raw, cell_idx, rows_tfp, cols_tfp, vals_tfp,
           rows_tfn, cols_tfn, vals_tfn, rows_ppp, cols_ppp, vals_ppp,
           rows_ppn, cols_ppn, vals_ppn, rows_und, cols_und, vals_und,
           rows_mir, cols_mir, vals_mir, g_tf_pos, g_tf_neg, g_ppi_pos,
           g_ppi_neg, g_undir, g_mirna_neg, alpha_logits, cell_emb,
           W_in, b_in, W_out, b_out):
    sp = jax.nn.softplus
    gains = [sp(g_tf_pos), -sp(g_tf_neg), sp(g_ppi_pos), -sp(g_ppi_neg),
             sp(g_undir), -sp(g_mirna_neg)]
    alphas = jax.nn.sigmoid(alpha_logits)

    rows_all = jnp.concatenate(
        [r.reshape(_NWG, _W) for r in (rows_tfp, rows_tfn, rows_ppp,
                                       rows_ppn, rows_und, rows_mir)]
        + [jnp.full((_NWPAD, _W), _PAD_ROW, jnp.int32)])
    cols_all = jnp.concatenate(
        [c.reshape(_NWG, _W) for c in (cols_tfp, cols_tfn, cols_ppp,
                                       cols_ppn, cols_und, cols_mir)]
        + [jnp.full((_NWPAD, _W), _PAD_COL, jnp.int32)])
    vals_all = jnp.concatenate(
        [(g * v).reshape(_NWG, _W)
         for g, v in zip(gains, (vals_tfp, vals_tfn, vals_ppp, vals_ppn,
                                 vals_und, vals_mir))]
        + [jnp.zeros((_NWPAD, _W), jnp.float32)]).reshape(_EPAD)

    u_pad = jnp.pad(u_raw, ((0, 0), (0, _NPAD - _N)))
    ctl_pad = jnp.pad(ctl_base, ((0, 0), (0, _NPAD - _N)))

    ab = jnp.concatenate([jnp.ones((1,), jnp.float32), alphas[:_STEPS - 1]])
    avec = jnp.stack([jnp.stack([jnp.full((16,), ab[k], jnp.float32),
                                 jnp.full((16,), 1.0 - ab[k], jnp.float32)])
                      for k in range(_STEPS)]).reshape(2 * _STEPS, 16)

    eflag, bflag = _zero_call()
    _, pbn = _prop_call(u_pad, avec, rows_all, cols_all, vals_all,
                        eflag, bflag)

    a5 = alphas[5]
    par = jnp.stack([a5, 1.0 - a5])
    bias_b = cell_emb[cell_idx] @ W_out[:, 0] + b_out[0]
    y = _mlp_call(ctl_pad, u_pad, pbn[0], pbn[1],
                  W_in, b_in, W_out[:, 0], par, bias_b[:, None])
    return y[:, :_N]
